# Initial kernel scaffold; baseline (speedup 1.0000x reference)
import jax
import jax.numpy as jnp
from jax.experimental import pallas as pl


def kernel(x, segmap, w1, b1, wg, bg, wb, bb):
    raise NotImplementedError("write your pallas kernel here")


# speedup vs baseline: 1.5129x; 1.5129x over previous
import functools

import jax
import jax.numpy as jnp
from jax.experimental import pallas as pl
from jax.experimental.pallas import tpu as pltpu

_EPS = 1e-5


def _fused_spade_kernel(seg_ref, m1_ref, m2_ref, w1_ref, b1_ref, wgb_ref,
                        bgb_ref, x_ref, o_ref, *, width, halo):
    npad = m1_ref.shape[-1]
    hw = x_ref.shape[-1]
    c = x_ref.shape[0]

    m1 = m1_ref[...]
    interior, nl1, nr1 = m1[0:1], m1[1:2], m1[2:3]

    taps1 = []
    for kh in range(3):
        for kw in range(3):
            off = (kh - 1) * width + (kw - 1)
            tap = seg_ref[:, halo + off: halo + off + npad]
            if kw == 0:
                tap = tap * nl1
            elif kw == 2:
                tap = tap * nr1
            taps1.append(tap)
    col1 = jnp.concatenate(taps1, axis=0)
    acc1 = jnp.dot(w1_ref[...], col1, preferred_element_type=jnp.float32)
    actv = jnp.maximum(acc1 + b1_ref[...], 0.0).astype(jnp.bfloat16)
    actv = actv * interior

    m2 = m2_ref[...]
    nl2, nr2 = m2[0:1], m2[1:2]
    taps2 = []
    for kh in range(3):
        for kw in range(3):
            off = (kh - 1) * width + (kw - 1)
            tap = actv[:, halo + off: halo + off + hw]
            if kw == 0:
                tap = tap * nl2
            elif kw == 2:
                tap = tap * nr2
            taps2.append(tap)
    col2 = jnp.concatenate(taps2, axis=0)
    gb = jnp.dot(wgb_ref[...], col2,
                 preferred_element_type=jnp.float32) + bgb_ref[...]

    x = x_ref[...]
    mean = jnp.mean(x, axis=-1, keepdims=True)
    diff = x - mean
    var = jnp.mean(diff * diff, axis=-1, keepdims=True)
    xhat = diff * jax.lax.rsqrt(var + _EPS)
    o_ref[...] = xhat * gb[:c, :] + gb[c:, :]


def _resize_nearest(seg, H, W):
    Hs, Ws = seg.shape[2], seg.shape[3]
    if (Hs, Ws) == (H, W):
        return seg
    seg = jnp.take(seg, (jnp.arange(H) * Hs) // H, axis=2)
    return jnp.take(seg, (jnp.arange(W) * Ws) // W, axis=3)


def kernel(x, segmap, w1, b1, wg, bg, wb, bb):
    B, C, H, W = x.shape
    nhidden, label_nc = w1.shape[0], w1.shape[1]
    HW = H * W
    halo = W + 1
    npad = HW + 2 * halo

    seg = _resize_nearest(segmap, H, W)
    seg_fp = jnp.pad(seg.reshape(B, label_nc, HW),
                     ((0, 0), (0, 0), (2 * halo, 2 * halo))
                     ).astype(jnp.bfloat16)
    x_flat = x.reshape(B, C, HW)

    p = jnp.arange(npad) - halo
    pw = p % W
    m1 = jnp.stack([(p >= 0) & (p < HW), pw != 0, pw != W - 1]
                   ).astype(jnp.bfloat16)
    j = jnp.arange(HW)
    m2 = jnp.stack([j % W != 0, j % W != W - 1]).astype(jnp.bfloat16)

    w1c = jnp.transpose(w1, (0, 2, 3, 1)).reshape(
        nhidden, 9 * label_nc).astype(jnp.bfloat16)
    wgbc = jnp.transpose(jnp.concatenate([wg, wb], axis=0),
                         (0, 2, 3, 1)).reshape(2 * C, 9 * nhidden
                                               ).astype(jnp.bfloat16)
    b1m = b1.reshape(nhidden, 1).astype(jnp.float32)
    bgbm = jnp.concatenate([1.0 + bg, bb]).reshape(2 * C, 1
                                                   ).astype(jnp.float32)

    out_flat = pl.pallas_call(
        functools.partial(_fused_spade_kernel, width=W, halo=halo),
        out_shape=jax.ShapeDtypeStruct((B, C, HW), x.dtype),
        grid=(B,),
        in_specs=[
            pl.BlockSpec((None, label_nc, npad + 2 * halo),
                         lambda b: (b, 0, 0)),
            pl.BlockSpec((3, npad), lambda b: (0, 0)),
            pl.BlockSpec((2, HW), lambda b: (0, 0)),
            pl.BlockSpec((nhidden, 9 * label_nc), lambda b: (0, 0)),
            pl.BlockSpec((nhidden, 1), lambda b: (0, 0)),
            pl.BlockSpec((2 * C, 9 * nhidden), lambda b: (0, 0)),
            pl.BlockSpec((2 * C, 1), lambda b: (0, 0)),
            pl.BlockSpec((None, C, HW), lambda b: (b, 0, 0)),
        ],
        out_specs=pl.BlockSpec((None, C, HW), lambda b: (b, 0, 0)),
        compiler_params=pltpu.CompilerParams(
            dimension_semantics=("parallel",),
            vmem_limit_bytes=60 * 1024 * 1024),
    )(seg_fp, m1, m2, w1c, b1m, wgbc, bgbm, x_flat)

    return out_flat.reshape(B, C, H, W)


# speedup vs baseline: 1.8276x; 1.2080x over previous
import functools

import jax
import jax.numpy as jnp
from jax.experimental import pallas as pl
from jax.experimental.pallas import tpu as pltpu

_EPS = 1e-5


def _fused_spade_kernel(seg_ref, m1_ref, m2_ref, w1_ref, b1_ref, wgb_ref,
                        bgb_ref, x_ref, o_ref, *, width, halo):
    npad = m1_ref.shape[-1]
    hw = x_ref.shape[-1]
    c = x_ref.shape[0]

    m1 = m1_ref[...]
    interior, nl1, nr1 = m1[0:1], m1[1:2], m1[2:3]

    taps1 = []
    for kh in range(3):
        for kw in range(3):
            off = (kh - 1) * width + (kw - 1)
            tap = seg_ref[:, halo + off: halo + off + npad]
            if kw == 0:
                tap = tap * nl1
            elif kw == 2:
                tap = tap * nr1
            taps1.append(tap)
    col1 = jnp.concatenate(taps1, axis=0)
    acc1 = jnp.dot(w1_ref[...], col1, preferred_element_type=jnp.float32)
    actv = jnp.maximum(acc1 + b1_ref[...], 0.0).astype(jnp.bfloat16)
    actv = actv * interior

    m2 = m2_ref[...]
    nl2, nr2 = m2[0:1], m2[1:2]
    taps2 = []
    for kh in range(3):
        for kw in range(3):
            off = (kh - 1) * width + (kw - 1)
            tap = actv[:, halo + off: halo + off + hw]
            if kw == 0:
                tap = tap * nl2
            elif kw == 2:
                tap = tap * nr2
            taps2.append(tap)
    col2 = jnp.concatenate(taps2, axis=0)
    gb = jnp.dot(wgb_ref[...], col2,
                 preferred_element_type=jnp.float32) + bgb_ref[...]

    x = x_ref[...]
    mean = jnp.mean(x, axis=-1, keepdims=True)
    diff = x - mean
    var = jnp.mean(diff * diff, axis=-1, keepdims=True)
    xhat = diff * jax.lax.rsqrt(var + _EPS)
    o_ref[...] = xhat * gb[:c, :] + gb[c:, :]


def _resize_nearest(seg, H, W):
    B, C, Hs, Ws = seg.shape
    if (Hs, Ws) == (H, W):
        return seg
    if H % Hs == 0 and W % Ws == 0:
        seg = jnp.broadcast_to(seg[:, :, :, None, :, None],
                               (B, C, Hs, H // Hs, Ws, W // Ws))
        return seg.reshape(B, C, H, W)
    seg = jnp.take(seg, (jnp.arange(H) * Hs) // H, axis=2)
    return jnp.take(seg, (jnp.arange(W) * Ws) // W, axis=3)


def kernel(x, segmap, w1, b1, wg, bg, wb, bb):
    B, C, H, W = x.shape
    nhidden, label_nc = w1.shape[0], w1.shape[1]
    HW = H * W
    halo = W + 1
    npad = HW + 2 * halo

    seg = _resize_nearest(segmap, H, W)
    seg_fp = jnp.pad(seg.reshape(B, label_nc, HW),
                     ((0, 0), (0, 0), (2 * halo, 2 * halo))
                     ).astype(jnp.bfloat16)
    x_flat = x.reshape(B, C, HW)

    p = jnp.arange(npad) - halo
    pw = p % W
    m1 = jnp.stack([(p >= 0) & (p < HW), pw != 0, pw != W - 1]
                   ).astype(jnp.bfloat16)
    j = jnp.arange(HW)
    m2 = jnp.stack([j % W != 0, j % W != W - 1]).astype(jnp.bfloat16)

    w1c = jnp.transpose(w1, (0, 2, 3, 1)).reshape(
        nhidden, 9 * label_nc).astype(jnp.bfloat16)
    wgbc = jnp.transpose(jnp.concatenate([wg, wb], axis=0),
                         (0, 2, 3, 1)).reshape(2 * C, 9 * nhidden
                                               ).astype(jnp.bfloat16)
    b1m = b1.reshape(nhidden, 1).astype(jnp.float32)
    bgbm = jnp.concatenate([1.0 + bg, bb]).reshape(2 * C, 1
                                                   ).astype(jnp.float32)

    out_flat = pl.pallas_call(
        functools.partial(_fused_spade_kernel, width=W, halo=halo),
        out_shape=jax.ShapeDtypeStruct((B, C, HW), x.dtype),
        grid=(B,),
        in_specs=[
            pl.BlockSpec((None, label_nc, npad + 2 * halo),
                         lambda b: (b, 0, 0)),
            pl.BlockSpec((3, npad), lambda b: (0, 0)),
            pl.BlockSpec((2, HW), lambda b: (0, 0)),
            pl.BlockSpec((nhidden, 9 * label_nc), lambda b: (0, 0)),
            pl.BlockSpec((nhidden, 1), lambda b: (0, 0)),
            pl.BlockSpec((2 * C, 9 * nhidden), lambda b: (0, 0)),
            pl.BlockSpec((2 * C, 1), lambda b: (0, 0)),
            pl.BlockSpec((None, C, HW), lambda b: (b, 0, 0)),
        ],
        out_specs=pl.BlockSpec((None, C, HW), lambda b: (b, 0, 0)),
        compiler_params=pltpu.CompilerParams(
            dimension_semantics=("parallel",),
            vmem_limit_bytes=60 * 1024 * 1024),
    )(seg_fp, m1, m2, w1c, b1m, wgbc, bgbm, x_flat)

    return out_flat.reshape(B, C, H, W)
